# Initial kernel scaffold; baseline (speedup 1.0000x reference)
import jax
import jax.numpy as jnp
from jax.experimental import pallas as pl


def kernel(node_embeddings, semalink_embeddings, hyperedge_embeddings, semalinks, W_lin, b_lin, W_r, W_sem, b_sem, ln_gamma, ln_beta):
    raise NotImplementedError("write your pallas kernel here")


# speedup vs baseline: 5.2843x; 5.2843x over previous
import functools

import jax
import jax.numpy as jnp
from jax import lax
from jax.experimental import pallas as pl
from jax.experimental.pallas import tpu as pltpu
from jax.experimental.pallas import tpu_sc as plsc

_INTERPRET = False

NC = 2
NS = 16
NW = NC * NS
L = 16
D = 128
DV = D // L


def _p0_body(ne_b, he_b, se_b, wr, wlt, wst, bsem, np_b, nl_b, rt_b, st_b):
    ne = ne_b[...]
    np_b[...] = jnp.dot(ne, wr[...], preferred_element_type=jnp.float32)
    nl_b[...] = jnp.dot(ne, wlt[...], preferred_element_type=jnp.float32)
    rt_b[...] = jnp.dot(he_b[...], wr[...], preferred_element_type=jnp.float32)
    st_b[...] = jnp.dot(se_b[...], wst[...], preferred_element_type=jnp.float32) + bsem[...]


def _p0(node_emb, he_emb, sem_emb, W_r, W_linT, W_semT, b_sem):
    n = node_emb.shape[0]
    blk = 1000 if n % 1000 == 0 else n
    grid = n // blk
    full = lambda s: pl.BlockSpec(s, lambda i: (0, 0))
    row = lambda w: pl.BlockSpec((blk, w), lambda i: (i, 0))
    return pl.pallas_call(
        _p0_body,
        grid=(grid,),
        in_specs=[row(D), row(D), row(sem_emb.shape[1]),
                  full(W_r.shape), full(W_linT.shape), full(W_semT.shape),
                  full(b_sem.shape)],
        out_specs=[row(D), row(D), row(D), row(D)],
        out_shape=[jax.ShapeDtypeStruct((n, D), jnp.float32)] * 4,
        interpret=_INTERPRET,
    )(node_emb, he_emb, sem_emb, W_r, W_linT, W_semT, b_sem)


def _s1_body(E, EPT, K, H,
             nidx, hidx, ridx, nproj, rmt, semt,
             scores, maxout,
             nvA, hvA, rvA, pn0, rt0, st0, pn1, rt1, st1, sbufA, maxtab,
             sem0, sem1):
    c = lax.axis_index("c")
    s = lax.axis_index("s")
    w = s * NC + c
    base = w * EPT
    lanes = lax.iota(jnp.int32, L)
    neg = jnp.full((L,), -jnp.inf, jnp.float32)
    NB = EPT // K

    def initb(i, _):
        maxtab[pl.ds(i * L, L)] = neg
        return 0
    lax.fori_loop(0, H // L, initb, 0, unroll=False)

    pltpu.sync_copy(nidx.at[pl.ds(base, EPT)], nvA)
    pltpu.sync_copy(hidx.at[pl.ds(base, EPT)], hvA)
    pltpu.sync_copy(ridx.at[pl.ds(base, EPT)], rvA)

    bufs = ((pn0, rt0, st0, sem0), (pn1, rt1, st1, sem1))

    def issue(bb, sl):
        pn, rt, st, sem = bufs[sl]
        pltpu.async_copy(nproj.at[nvA.at[pl.ds(bb * K, K)]], pn, sem)
        pltpu.async_copy(rmt.at[hvA.at[pl.ds(bb * K, K)]], rt, sem)
        pltpu.async_copy(semt.at[rvA.at[pl.ds(bb * K, K)]], st, sem)

    def drain(sl):
        pn, rt, st, sem = bufs[sl]
        pltpu.make_async_copy(nproj.at[pl.ds(0, K)], pn, sem).wait()
        pltpu.make_async_copy(rmt.at[pl.ds(0, K)], rt, sem).wait()
        pltpu.make_async_copy(semt.at[pl.ds(0, K)], st, sem).wait()

    def compute(bb, sl):
        pn, rt, st, sem = bufs[sl]

        def edge(j, _):
            accs = []
            for dd in range(DV):
                a = rt[j, pl.ds(dd * L, L)]
                bbv = st[j, pl.ds(dd * L, L)]
                pv = pn[j, pl.ds(dd * L, L)]
                x = a + bbv
                e2 = jnp.exp(x + x)
                th = 1.0 - 2.0 / (e2 + 1.0)
                accs.append(pv * th)
            s8 = ((accs[0] + accs[1]) + (accs[2] + accs[3])) + (
                (accs[4] + accs[5]) + (accs[6] + accs[7]))
            sc = jnp.sum(s8)
            plsc.store_scatter(sbufA, [jnp.full((L,), bb * K + j, jnp.int32)],
                               jnp.full((L,), sc, jnp.float32),
                               mask=lanes == 0)
            return 0
        lax.fori_loop(0, K, edge, 0, unroll=2)

        def grp(g, _):
            hvv = hvA[pl.ds(bb * K + g * L, L)]
            svv = sbufA[pl.ds(bb * K + g * L, L)]

            def cond_fn(_c):
                cur = plsc.load_gather(maxtab, [hvv])
                return jnp.any(svv > cur)

            def body_fn(_c):
                cur = plsc.load_gather(maxtab, [hvv])
                plsc.store_scatter(maxtab, [hvv], svv, mask=svv > cur)
                return 0
            lax.while_loop(cond_fn, body_fn, 0)
            return 0
        lax.fori_loop(0, K // L, grp, 0, unroll=False)

    issue(0, 0)
    issue(1, 1)

    def pair(i, _):
        for sl in range(2):
            bb = 2 * i + sl

            @pl.when(bb < NB)
            def _():
                drain(sl)
                compute(bb, sl)

                @pl.when(bb + 2 < NB)
                def _():
                    issue(bb + 2, sl)
        return 0
    lax.fori_loop(0, (NB + 1) // 2, pair, 0, unroll=False)

    pltpu.sync_copy(sbufA, scores.at[pl.ds(base, EPT)])
    pltpu.sync_copy(maxtab, maxout.at[w])


def _s1(nidx, hidx, ridx, nproj, rmt, semt):
    E = nidx.shape[0]
    H = rmt.shape[0]
    EPT = E // NW
    K = 80 if EPT % 80 == 0 else EPT
    mesh = plsc.VectorSubcoreMesh(core_axis_name="c", subcore_axis_name="s", num_cores=NC, num_subcores=NS)
    f = pl.kernel(
        functools.partial(_s1_body, E, EPT, K, H),
        out_type=[jax.ShapeDtypeStruct((E,), jnp.float32),
                  jax.ShapeDtypeStruct((NW, H), jnp.float32)],
        mesh=mesh,
        compiler_params=pltpu.CompilerParams(needs_layout_passes=False, use_tc_tiling_on_sc=False),
        scratch_types=[
            pltpu.VMEM((EPT,), jnp.int32),
            pltpu.VMEM((EPT,), jnp.int32),
            pltpu.VMEM((EPT,), jnp.int32),
            pltpu.VMEM((K, D), jnp.float32),
            pltpu.VMEM((K, D), jnp.float32),
            pltpu.VMEM((K, D), jnp.float32),
            pltpu.VMEM((K, D), jnp.float32),
            pltpu.VMEM((K, D), jnp.float32),
            pltpu.VMEM((K, D), jnp.float32),
            pltpu.VMEM((EPT,), jnp.float32),
            pltpu.VMEM((H,), jnp.float32),
            pltpu.SemaphoreType.DMA,
            pltpu.SemaphoreType.DMA,
        ],
        interpret=_INTERPRET,
    )
    return f(nidx, hidx, ridx, nproj, rmt, semt)


def _p2_body(mx_b, c_b):
    m = jnp.max(mx_b[...], axis=0, keepdims=True)
    c_b[...] = jnp.where(jnp.isfinite(m), m, 0.0)


def _p2(maxout):
    H = maxout.shape[1]
    return pl.pallas_call(
        _p2_body,
        out_shape=jax.ShapeDtypeStruct((1, H), jnp.float32),
        interpret=_INTERPRET,
    )(maxout)


AW = 144


def _s3_body(E, EPT, K, H, HA,
             nidx, hidx, scores, cvec, nlin,
             out0, out1,
             nv0, nv1, hv0, hv1, sv0, sv1, hs0, hs1, exb,
             rows0, rows1, outb0, outb1, ctab, zbuf,
             isem0, isem1, gsem0, gsem1, ssem0, ssem1, accS):
    c = lax.axis_index("c")
    s = lax.axis_index("s")
    base = s * EPT
    lanes = lax.iota(jnp.int32, L)
    H2 = H // 2
    hbase = c * H2
    rpt = HA // NS
    NB = EPT // K
    idxb = ((nv0, hv0, sv0, isem0), (nv1, hv1, sv1, isem1))
    rowb = ((rows0, gsem0), (rows1, gsem1))
    scatb = ((outb0, hs0, ssem0), (outb1, hs1, ssem1))

    pltpu.sync_copy(cvec.at[0], ctab)

    zv = jnp.zeros((L,), jnp.float32)
    zrows = zbuf.shape[0]

    def zinit(i, _):
        for kk in range(AW // L):
            zbuf[i, pl.ds(kk * L, L)] = zv
        return 0
    lax.fori_loop(0, zrows, zinit, 0, unroll=False)

    def zcp(i, _):
        pltpu.sync_copy(zbuf, accS.at[pl.ds(s * rpt + i * zrows, zrows)])
        return 0
    lax.fori_loop(0, rpt // zrows, zcp, 0, unroll=False)
    plsc.subcore_barrier()

    def issue_idx(bb, sl):
        nv, hv, sv, isem = idxb[sl]
        pltpu.async_copy(nidx.at[pl.ds(base + bb * K, K)], nv, isem)
        pltpu.async_copy(hidx.at[pl.ds(base + bb * K, K)], hv, isem)
        pltpu.async_copy(scores.at[pl.ds(base + bb * K, K)], sv, isem)

    def drain_idx(sl):
        nv, hv, sv, isem = idxb[sl]
        pltpu.make_async_copy(nidx.at[pl.ds(0, K)], nv, isem).wait()
        pltpu.make_async_copy(hidx.at[pl.ds(0, K)], hv, isem).wait()
        pltpu.make_async_copy(scores.at[pl.ds(0, K)], sv, isem).wait()

    def issue_rows(sl):
        nv = idxb[sl][0]
        rows, gsem = rowb[sl]
        pltpu.async_copy(nlin.at[nv], rows, gsem)

    def drain_rows(sl):
        rows, gsem = rowb[sl]
        pltpu.make_async_copy(nlin.at[pl.ds(0, K)], rows, gsem).wait()

    def issue_scat(sl):
        outb, hs, ssem = scatb[sl]
        pltpu.async_copy(outb, accS.at[hs], ssem, add=True)

    def drain_scat(sl):
        outb, hs, ssem = scatb[sl]
        pltpu.make_async_copy(out0.at[pl.ds(0, K)], outb, ssem).wait()

    def compute(bb, sl):
        nv, hv, sv, isem = idxb[sl]
        rows, gsem = rowb[sl]
        outb, hs, ssem = scatb[sl]

        def grpfn(g, _):
            hvv = hv[pl.ds(g * L, L)]
            svv = sv[pl.ds(g * L, L)]
            cv = plsc.load_gather(ctab, [hvv])
            exb[pl.ds(g * L, L)] = jnp.exp(svv - cv)
            return 0
        lax.fori_loop(0, K // L, grpfn, 0, unroll=False)
        drain_rows(sl)

        def edge(j, _):
            exj = plsc.load_gather(exb, [jnp.full((L,), j, jnp.int32)])
            for dd in range(DV):
                outb[j, pl.ds(dd * L, L)] = rows[j, pl.ds(dd * L, L)] * exj
            outb[j, pl.ds(D, L)] = jnp.where(lanes == 0, exj, 0.0)
            return 0
        lax.fori_loop(0, K, edge, 0, unroll=4)
        for g in range(K // L):
            hvv = hv[pl.ds(g * L, L)]
            ih = hvv - hbase
            valid = (ih >= 0) & (ih < H2)
            hs[pl.ds(g * L, L)] = jnp.where(valid, ih, H2)

    issue_idx(0, 0)
    issue_idx(1, 1)
    drain_idx(0)
    issue_rows(0)

    def pair(i, _):
        for sl in range(2):
            bb = 2 * i + sl
            osl = 1 - sl

            @pl.when(bb < NB)
            def _():
                @pl.when(bb + 1 < NB)
                def _():
                    drain_idx(osl)
                    issue_rows(osl)

                @pl.when(bb >= 2)
                def _():
                    drain_scat(sl)
                compute(bb, sl)
                issue_scat(sl)

                @pl.when(bb + 2 < NB)
                def _():
                    issue_idx(bb + 2, sl)
        return 0
    lax.fori_loop(0, (NB + 1) // 2, pair, 0, unroll=False)

    drain_scat((NB - 2) % 2)
    drain_scat((NB - 1) % 2)
    plsc.subcore_barrier()

    @pl.when(c == 0)
    def _():
        pltpu.sync_copy(accS.at[pl.ds(s * rpt, rpt)], out0.at[pl.ds(s * rpt, rpt)])

    @pl.when(c == 1)
    def _():
        pltpu.sync_copy(accS.at[pl.ds(s * rpt, rpt)], out1.at[pl.ds(s * rpt, rpt)])


def _s3(nidx, hidx, scores, cvec, nlin):
    E = nidx.shape[0]
    H = cvec.shape[1]
    EPT = E // NS
    K = 32 if EPT % 32 == 0 else EPT
    HA = (H // 2 + 16 + NS * 16 - 1) // (NS * 16) * (NS * 16)
    zrows = 16 if (HA // NS) % 16 == 0 else HA // NS
    mesh = plsc.VectorSubcoreMesh(core_axis_name="c", subcore_axis_name="s", num_cores=NC, num_subcores=NS)
    f = pl.kernel(
        functools.partial(_s3_body, E, EPT, K, H, HA),
        out_type=[jax.ShapeDtypeStruct((HA, AW), jnp.float32),
                  jax.ShapeDtypeStruct((HA, AW), jnp.float32)],
        mesh=mesh,
        compiler_params=pltpu.CompilerParams(needs_layout_passes=False, use_tc_tiling_on_sc=False),
        scratch_types=[
            pltpu.VMEM((K,), jnp.int32),
            pltpu.VMEM((K,), jnp.int32),
            pltpu.VMEM((K,), jnp.int32),
            pltpu.VMEM((K,), jnp.int32),
            pltpu.VMEM((K,), jnp.float32),
            pltpu.VMEM((K,), jnp.float32),
            pltpu.VMEM((K,), jnp.int32),
            pltpu.VMEM((K,), jnp.int32),
            pltpu.VMEM((K,), jnp.float32),
            pltpu.VMEM((K, D), jnp.float32),
            pltpu.VMEM((K, D), jnp.float32),
            pltpu.VMEM((K, AW), jnp.float32),
            pltpu.VMEM((K, AW), jnp.float32),
            pltpu.VMEM((H,), jnp.float32),
            pltpu.VMEM((zrows, AW), jnp.float32),
            pltpu.SemaphoreType.DMA,
            pltpu.SemaphoreType.DMA,
            pltpu.SemaphoreType.DMA,
            pltpu.SemaphoreType.DMA,
            pltpu.SemaphoreType.DMA,
            pltpu.SemaphoreType.DMA,
            pltpu.VMEM_SHARED((HA, AW), jnp.float32),
        ],
        interpret=_INTERPRET,
    )
    return f(nidx, hidx, scores, cvec, nlin)


def _p4_body(a_b, blin, gam, bet, out_b):
    x = a_b[...]
    num = x[:, :D]
    den = x[:, D:D + 1]
    y = num / jnp.maximum(den, 1e-16) + blin[...]
    y = jnp.where(y > 0, y, 0.01 * y)
    mu = jnp.mean(y, axis=1, keepdims=True)
    var = jnp.mean((y - mu) ** 2, axis=1, keepdims=True)
    out_b[...] = (y - mu) * lax.rsqrt(var + 1e-5) * gam[...] + bet[...]


def _p4(acc, b_lin, gamma, beta):
    H = acc.shape[0]
    blk = 1000 if H % 1000 == 0 else H
    grid = H // blk
    row = lambda w: pl.BlockSpec((blk, w), lambda i: (i, 0))
    full = lambda s: pl.BlockSpec(s, lambda i: (0, 0))
    return pl.pallas_call(
        _p4_body,
        grid=(grid,),
        in_specs=[row(AW), full(b_lin.shape), full(gamma.shape),
                  full(beta.shape)],
        out_specs=row(D),
        out_shape=jax.ShapeDtypeStruct((H, D), jnp.float32),
        interpret=_INTERPRET,
    )(acc, b_lin, gamma, beta)


def kernel(node_embeddings, semalink_embeddings, hyperedge_embeddings, semalinks,
           W_lin, b_lin, W_r, W_sem, b_sem, ln_gamma, ln_beta):
    nidx = semalinks[:, 0]
    hidx = semalinks[:, 1]
    ridx = semalinks[:, 2]

    nproj, nlin, rmt, semt = _p0(
        node_embeddings, hyperedge_embeddings, semalink_embeddings,
        W_r, W_lin.T, W_sem.T, b_sem.reshape(1, -1))

    scores, maxout = _s1(nidx, hidx, ridx, nproj, rmt, semt)
    cvec = _p2(maxout)
    acc0, acc1 = _s3(nidx, hidx, scores, cvec, nlin)
    H = hyperedge_embeddings.shape[0]
    acc = jnp.concatenate([acc0[:H // 2], acc1[:H // 2]], axis=0)
    return _p4(acc, b_lin.reshape(1, -1), ln_gamma.reshape(1, -1),
               ln_beta.reshape(1, -1))
